# baseline (speedup 1.0000x reference)
import functools

import jax
import jax.numpy as jnp
from jax import lax
from jax.experimental import pallas as pl
from jax.experimental.pallas import tpu as pltpu
from jax.experimental.pallas import tpu_sc as plsc

_NUM_WORKERS = 32
_CHUNK = 320
_NB = 8


@functools.lru_cache(maxsize=None)
def _make_gather(B, D, chunk):
    b_per_w = B // _NUM_WORKERS
    n_chunks = b_per_w // chunk
    mesh = plsc.VectorSubcoreMesh(core_axis_name="c", subcore_axis_name="s")

    @functools.partial(
        pl.kernel,
        mesh=mesh,
        out_type=(
            jax.ShapeDtypeStruct((B, D), jnp.float32),
            jax.ShapeDtypeStruct((B, D), jnp.float32),
        ),
        scratch_types=[
            pltpu.VMEM((b_per_w,), jnp.int32),
            pltpu.VMEM((2, chunk, D), jnp.float32),
            pltpu.VMEM((2, chunk, D), jnp.float32),
            pltpu.SemaphoreType.DMA((2,)),
        ],
        compiler_params=pltpu.CompilerParams(use_tc_tiling_on_sc=False),
    )
    def gather_kernel(amp_hbm, phase_hbm, idx_hbm, amp_out, phase_out,
                      idx_v, amp_v, phase_v, gsem):
        wid = lax.axis_index("s") * 2 + lax.axis_index("c")
        base0 = wid * b_per_w
        pltpu.sync_copy(idx_hbm.at[pl.ds(base0, b_per_w)], idx_v)

        def fire(r, b):
            idx_slice = idx_v.at[pl.ds(r * chunk, chunk)]
            pltpu.async_copy(amp_hbm.at[idx_slice], amp_v.at[b], gsem.at[b])
            pltpu.async_copy(phase_hbm.at[idx_slice], phase_v.at[b],
                             gsem.at[b])

        def drain(r, b):
            idx_slice = idx_v.at[pl.ds(r * chunk, chunk)]
            pltpu.make_async_copy(amp_hbm.at[idx_slice], amp_v.at[b],
                                  gsem.at[b]).wait()
            pltpu.make_async_copy(phase_hbm.at[idx_slice], phase_v.at[b],
                                  gsem.at[b]).wait()

        fire(0, 0)

        def body(g, carry):
            for b in range(2):
                r = g * 2 + b
                nb = (b + 1) % 2
                if b == 0:
                    fire(r + 1, nb)
                else:
                    @pl.when(g < n_chunks // 2 - 1)
                    def _():
                        fire(r + 1, nb)
                drain(r, b)
                out_base = base0 + r * chunk
                pltpu.sync_copy(amp_v.at[b], amp_out.at[pl.ds(out_base,
                                                              chunk)])
                pltpu.sync_copy(phase_v.at[b],
                                phase_out.at[pl.ds(out_base, chunk)])
            return carry

        lax.fori_loop(0, n_chunks // 2, body, 0)

    return gather_kernel


@functools.lru_cache(maxsize=None)
def _make_repack(BATCH, HIST, D):
    rp_b = HIST * D // 128

    def repack_body(a_ref, p_ref, ao_ref, po_ref):
        for q in range(rp_b):
            rows = pl.Slice(q, _NB, rp_b)
            xa = a_ref[rows, :]
            xp = p_ref[rows, :]
            ao_ref[:, 2 * q, :] = xa[:, :D]
            ao_ref[:, 2 * q + 1, :] = xa[:, D:]
            po_ref[:, 2 * q, :] = xp[:, :D]
            po_ref[:, 2 * q + 1, :] = xp[:, D:]

    return pl.pallas_call(
        repack_body,
        grid=(BATCH // _NB,),
        in_specs=[
            pl.BlockSpec((_NB * rp_b, 128), lambda i: (i, 0)),
            pl.BlockSpec((_NB * rp_b, 128), lambda i: (i, 0)),
        ],
        out_specs=[
            pl.BlockSpec((_NB, HIST, D), lambda i: (i, 0, 0)),
            pl.BlockSpec((_NB, HIST, D), lambda i: (i, 0, 0)),
        ],
        out_shape=(
            jax.ShapeDtypeStruct((BATCH, HIST, D), jnp.float32),
            jax.ShapeDtypeStruct((BATCH, HIST, D), jnp.float32),
        ),
    )


def kernel(amplitude_table, phase_table, indices):
    batch, hist = indices.shape
    d = amplitude_table.shape[1]
    b_total = batch * hist
    idx_flat = indices.reshape(b_total).astype(jnp.int32)
    g = _make_gather(b_total, d, _CHUNK)
    amp_flat, ph_flat = g(amplitude_table, phase_table, idx_flat)
    amp128 = amp_flat.reshape(b_total * d // 128, 128)
    ph128 = ph_flat.reshape(b_total * d // 128, 128)
    rp = _make_repack(batch, hist, d)
    return rp(amp128, ph128)


# speedup vs baseline: 1.5775x; 1.5775x over previous
import functools

import jax
import jax.numpy as jnp
from jax import lax
from jax.experimental import pallas as pl
from jax.experimental.pallas import tpu as pltpu
from jax.experimental.pallas import tpu_sc as plsc

_NUM_WORKERS = 32
_CHUNK = 160
_NBUF = 4
_AHEAD = 2


@functools.lru_cache(maxsize=None)
def _make_kernel(B, D, chunk):
    b_per_w = B // _NUM_WORKERS
    n_chunks = b_per_w // chunk
    n_outer = n_chunks // _NBUF
    mesh = plsc.VectorSubcoreMesh(core_axis_name="c", subcore_axis_name="s")

    @functools.partial(
        pl.kernel,
        mesh=mesh,
        out_type=(
            jax.ShapeDtypeStruct((B, D), jnp.float32),
            jax.ShapeDtypeStruct((B, D), jnp.float32),
        ),
        scratch_types=[
            pltpu.VMEM((b_per_w,), jnp.int32),
            pltpu.VMEM((_NBUF, chunk, D), jnp.float32),
            pltpu.VMEM((_NBUF, chunk, D), jnp.float32),
            pltpu.SemaphoreType.DMA((_NBUF,)),
            pltpu.SemaphoreType.DMA((_NBUF,)),
        ],
        compiler_params=pltpu.CompilerParams(use_tc_tiling_on_sc=False),
    )
    def gather_kernel(amp_hbm, phase_hbm, idx_hbm, amp_out, phase_out,
                      idx_v, amp_v, phase_v, gsem, wsem):
        wid = lax.axis_index("s") * 2 + lax.axis_index("c")
        base0 = wid * b_per_w
        pltpu.sync_copy(idx_hbm.at[pl.ds(base0, b_per_w)], idx_v)

        def gather_descs(r, b):
            idx_slice = idx_v.at[pl.ds(r * chunk, chunk)]
            return (
                pltpu.make_async_copy(amp_hbm.at[idx_slice], amp_v.at[b],
                                      gsem.at[b]),
                pltpu.make_async_copy(phase_hbm.at[idx_slice], phase_v.at[b],
                                      gsem.at[b]),
            )

        def write_descs(r, b):
            out_base = base0 + r * chunk
            return (
                pltpu.make_async_copy(amp_v.at[b],
                                      amp_out.at[pl.ds(out_base, chunk)],
                                      wsem.at[b]),
                pltpu.make_async_copy(phase_v.at[b],
                                      phase_out.at[pl.ds(out_base, chunk)],
                                      wsem.at[b]),
            )

        def fire(descs):
            for d in descs:
                d.start()

        def drain(descs):
            for d in descs:
                d.wait()

        for r0 in range(_AHEAD):
            fire(gather_descs(r0, r0))

        def body(g, carry):
            for b in range(_NBUF):
                i = g * _NBUF + b
                fb = (b + _AHEAD) % _NBUF
                fi = i + _AHEAD

                @pl.when(fi < n_chunks)
                def _():
                    @pl.when(fi >= _NBUF)
                    def _():
                        drain(write_descs(fi - _NBUF, fb))
                    fire(gather_descs(fi, fb))

                drain(gather_descs(i, b))
                fire(write_descs(i, b))
            return carry

        lax.fori_loop(0, n_outer, body, 0)

        for b in range(_NBUF):
            drain(write_descs(n_chunks - _NBUF + b, b))

    return gather_kernel


def kernel(amplitude_table, phase_table, indices):
    batch, hist = indices.shape
    d = amplitude_table.shape[1]
    b_total = batch * hist
    idx_flat = indices.reshape(b_total).astype(jnp.int32)
    k = _make_kernel(b_total, d, _CHUNK)
    amp, ph = k(amplitude_table, phase_table, idx_flat)
    return amp.reshape(batch, hist, d), ph.reshape(batch, hist, d)
